# baseline (speedup 1.0000x reference)
import functools

import jax
import jax.numpy as jnp
from jax import lax
from jax.experimental import pallas as pl
from jax.experimental.pallas import tpu as pltpu
from jax.experimental.pallas import tpu_sc as plsc

N = 10000
E = 320000
F_IN = 128
F_HID = 128
F_OUT = 40
F_OUT_PAD = 128

NC = 2
NS = 16
NW = NC * NS
EDGES_PER_TILE = E // NW
CHUNK = 128
NFULL = EDGES_PER_TILE // CHUNK
REM = EDGES_PER_TILE - NFULL * CHUNK
ACH = 128
BASE_CH = 80
BLK = 40
NBLKI = BASE_CH // BLK
TOTAL_CHUNKS = BASE_CH * NW
PACK_SHIFT = 14
N_PAD = N + 8
ZROWS = 624
ZROWS_LAST = N - (NS - 1) * ZROWS
ZROWS_LAST_PAD = N_PAD - (NS - 1) * ZROWS

_MESH = plsc.VectorSubcoreMesh(
    core_axis_name="c", subcore_axis_name="s", num_cores=NC, num_subcores=NS
)


@functools.partial(
    pl.kernel,
    out_type=jax.ShapeDtypeStruct((NC, 2, N), jnp.float32),
    mesh=_MESH,
    scratch_types=[
        pltpu.VMEM((CHUNK,), jnp.int32),
        pltpu.VMEM((REM,), jnp.int32),
        pltpu.VMEM((CHUNK,), jnp.float32),
        pltpu.VMEM_SHARED((N,), jnp.float32),
        pltpu.VMEM_SHARED((N,), jnp.float32),
    ],
)
def _sc_degrees(src_hbm, dst_hbm, zeros_hbm, out_hbm,
                idx_v, idx_r, ones_v, acc_out, acc_in):
    c = lax.axis_index("c")
    s = lax.axis_index("s")
    tid = c * NS + s
    for j in range(CHUNK // 16):
        ones_v[pl.ds(j * 16, 16)] = jnp.full((16,), 1.0, jnp.float32)

    @pl.when(s == 0)
    def _():
        pltpu.sync_copy(zeros_hbm, acc_out)
        pltpu.sync_copy(zeros_hbm, acc_in)

    plsc.subcore_barrier()
    base = tid * EDGES_PER_TILE

    def body(i, carry):
        e0 = base + i * CHUNK
        pltpu.sync_copy(src_hbm.at[pl.ds(e0, CHUNK)], idx_v)
        pltpu.sync_copy(ones_v, acc_out.at[idx_v], add=True)
        pltpu.sync_copy(dst_hbm.at[pl.ds(e0, CHUNK)], idx_v)
        pltpu.sync_copy(ones_v, acc_in.at[idx_v], add=True)
        return carry

    lax.fori_loop(0, NFULL, body, 0)
    e0 = base + NFULL * CHUNK
    pltpu.sync_copy(src_hbm.at[pl.ds(e0, REM)], idx_r)
    pltpu.sync_copy(ones_v.at[pl.ds(0, REM)], acc_out.at[idx_r], add=True)
    pltpu.sync_copy(dst_hbm.at[pl.ds(e0, REM)], idx_r)
    pltpu.sync_copy(ones_v.at[pl.ds(0, REM)], acc_in.at[idx_r], add=True)
    plsc.subcore_barrier()

    @pl.when(s == 0)
    def _():
        pltpu.sync_copy(acc_out, out_hbm.at[c, 0])
        pltpu.sync_copy(acc_in, out_hbm.at[c, 1])


def _make_sc_agg(D):
    @functools.partial(
        pl.kernel,
        out_type=jax.ShapeDtypeStruct((NC, N, D), jnp.float32),
        mesh=_MESH,
        scratch_types=[
            pltpu.VMEM((ACH,), jnp.int32),
            pltpu.VMEM((ACH,), jnp.int32),
            pltpu.VMEM((ACH, D), jnp.float32),
            pltpu.VMEM_SHARED((N_PAD, D), jnp.float32),
        ],
    )
    def _agg(x_hbm, srcp_hbm, dstp_hbm, zeros_hbm, out_hbm,
             idx_s, idx_d, buf, acc):
        c = lax.axis_index("c")
        s = lax.axis_index("s")
        tid = c * NS + s

        @pl.when(s < NS - 1)
        def _():
            pltpu.sync_copy(zeros_hbm.at[pl.ds(s * ZROWS, ZROWS)],
                            acc.at[pl.ds(s * ZROWS, ZROWS)])

        @pl.when(s == NS - 1)
        def _():
            pltpu.sync_copy(zeros_hbm.at[pl.ds((NS - 1) * ZROWS, ZROWS_LAST_PAD)],
                            acc.at[pl.ds((NS - 1) * ZROWS, ZROWS_LAST_PAD)])

        plsc.subcore_barrier()
        ebase = tid * BASE_CH * ACH

        def body(j, carry):
            e0 = ebase + j * ACH
            pltpu.sync_copy(srcp_hbm.at[pl.ds(e0, ACH)], idx_s)
            pltpu.sync_copy(x_hbm.at[idx_s], buf)
            pltpu.sync_copy(dstp_hbm.at[pl.ds(e0, ACH)], idx_d)
            pltpu.sync_copy(buf, acc.at[idx_d], add=True)
            return carry

        lax.fori_loop(0, BASE_CH, body, 0)
        plsc.subcore_barrier()

        @pl.when(s < NS - 1)
        def _():
            pltpu.sync_copy(acc.at[pl.ds(s * ZROWS, ZROWS)],
                            out_hbm.at[c, pl.ds(s * ZROWS, ZROWS)])

        @pl.when(s == NS - 1)
        def _():
            pltpu.sync_copy(acc.at[pl.ds((NS - 1) * ZROWS, ZROWS_LAST)],
                            out_hbm.at[c, pl.ds((NS - 1) * ZROWS, ZROWS_LAST)])

    return _agg


_sc_agg128 = _make_sc_agg(F_HID)
_sc_agg48 = _make_sc_agg(F_OUT_PAD)


BN = 1000


def _tc_norms(degp):
    def body(d_ref, o_ref):
        d = d_ref[0] + d_ref[1]
        o_ref[...] = lax.rsqrt(jnp.where(d > 0, d, 1.0))

    return pl.pallas_call(
        body,
        out_shape=jax.ShapeDtypeStruct((2, N), jnp.float32),
    )(degp)


def _tc_layer0(feat, ns, w):
    def body(f_ref, ns_ref, w_ref, o_ref):
        o_ref[...] = jnp.dot(f_ref[...] * ns_ref[...], w_ref[...],
                             preferred_element_type=jnp.float32)

    return pl.pallas_call(
        body,
        grid=(N // BN,),
        in_specs=[
            pl.BlockSpec((BN, F_IN), lambda i: (i, 0)),
            pl.BlockSpec((BN, 1), lambda i: (i, 0)),
            pl.BlockSpec((F_IN, F_HID), lambda i: (0, 0)),
        ],
        out_specs=pl.BlockSpec((BN, F_HID), lambda i: (i, 0)),
        out_shape=jax.ShapeDtypeStruct((N, F_HID), jnp.float32),
    )(feat, ns, w)


def _tc_mid(aggp, nd, b, ns, w, d_out):
    def body(a_ref, nd_ref, b_ref, ns_ref, w_ref, o_ref):
        a = a_ref[0] + a_ref[1]
        h = jnp.maximum(a * nd_ref[...] + b_ref[...], 0.0)
        o_ref[...] = jnp.dot(h * ns_ref[...], w_ref[...],
                             preferred_element_type=jnp.float32)

    return pl.pallas_call(
        body,
        grid=(N // BN,),
        in_specs=[
            pl.BlockSpec((NC, BN, F_HID), lambda i: (0, i, 0)),
            pl.BlockSpec((BN, 1), lambda i: (i, 0)),
            pl.BlockSpec((1, F_HID), lambda i: (0, 0)),
            pl.BlockSpec((BN, 1), lambda i: (i, 0)),
            pl.BlockSpec((F_HID, d_out), lambda i: (0, 0)),
        ],
        out_specs=pl.BlockSpec((BN, d_out), lambda i: (i, 0)),
        out_shape=jax.ShapeDtypeStruct((N, d_out), jnp.float32),
    )(aggp, nd, b, ns, w)


def _tc_final(aggp, nd, b):
    def body(a_ref, nd_ref, b_ref, o_ref):
        a = a_ref[0] + a_ref[1]
        o_ref[...] = a * nd_ref[...] + b_ref[...]

    return pl.pallas_call(
        body,
        grid=(N // BN,),
        in_specs=[
            pl.BlockSpec((NC, BN, F_OUT_PAD), lambda i: (0, i, 0)),
            pl.BlockSpec((BN, 1), lambda i: (i, 0)),
            pl.BlockSpec((1, F_OUT_PAD), lambda i: (0, 0)),
        ],
        out_specs=pl.BlockSpec((BN, F_OUT_PAD), lambda i: (i, 0)),
        out_shape=jax.ShapeDtypeStruct((N, F_OUT_PAD), jnp.float32),
    )(aggp, nd, b)


def kernel(feat, edge_index, W0, b0, Wh, bh, Wo, bo):
    src = edge_index[0]
    dst = edge_index[1]
    pad_e = TOTAL_CHUNKS * ACH - E
    srcp = jnp.concatenate([src, jnp.zeros((pad_e,), jnp.int32)])
    dstp = jnp.concatenate([dst, jnp.full((pad_e,), N, jnp.int32)])
    zeros_n = jnp.zeros((N,), jnp.float32)
    zeros128 = jnp.zeros((N_PAD, F_HID), jnp.float32)
    zeros48 = jnp.zeros((N_PAD, F_OUT_PAD), jnp.float32)

    degp = _sc_degrees(src, dst, zeros_n)
    norms = _tc_norms(degp)
    ns = norms[0].reshape(N, 1)
    nd = norms[1].reshape(N, 1)

    x0 = _tc_layer0(feat, ns, W0)
    a0 = _sc_agg128(x0, srcp, dstp, zeros128)
    x1 = _tc_mid(a0, nd, b0.reshape(1, F_HID), ns, Wh, F_HID)
    a1 = _sc_agg128(x1, srcp, dstp, zeros128)

    wo_p = jnp.zeros((F_HID, F_OUT_PAD), jnp.float32).at[:, :F_OUT].set(Wo)
    bo_p = jnp.zeros((1, F_OUT_PAD), jnp.float32).at[0, :F_OUT].set(bo)
    x2 = _tc_mid(a1, nd, bh.reshape(1, F_HID), ns, wo_p, F_OUT_PAD)
    a2 = _sc_agg48(x2, srcp, dstp, zeros48)
    out = _tc_final(a2, nd, bo_p)
    return out[:, :F_OUT]


# speedup vs baseline: 1.2710x; 1.2710x over previous
import functools

import jax
import jax.numpy as jnp
from jax import lax
from jax.experimental import pallas as pl
from jax.experimental.pallas import tpu as pltpu
from jax.experimental.pallas import tpu_sc as plsc

N = 10000
E = 320000
F_IN = 128
F_HID = 128
F_OUT = 40
F_OUT_PAD = 128

NC = 2
NS = 16
NW = NC * NS
EDGES_PER_TILE = E // NW
CHUNK = 128
NFULL = EDGES_PER_TILE // CHUNK
REM = EDGES_PER_TILE - NFULL * CHUNK
ACH = 128
BASE_CH = 80
TOTAL_CHUNKS = BASE_CH * NW
PACK_SHIFT = 14
N_PAD = N + 8
ZROWS = 624
ZROWS_LAST = N - (NS - 1) * ZROWS
ZROWS_LAST_PAD = N_PAD - (NS - 1) * ZROWS

_MESH = plsc.VectorSubcoreMesh(
    core_axis_name="c", subcore_axis_name="s", num_cores=NC, num_subcores=NS
)


@functools.partial(
    pl.kernel,
    out_type=jax.ShapeDtypeStruct((NC, 2, N), jnp.float32),
    mesh=_MESH,
    scratch_types=[
        pltpu.VMEM((CHUNK,), jnp.int32),
        pltpu.VMEM((REM,), jnp.int32),
        pltpu.VMEM((CHUNK,), jnp.float32),
        pltpu.VMEM_SHARED((N,), jnp.float32),
        pltpu.VMEM_SHARED((N,), jnp.float32),
    ],
)
def _sc_degrees(src_hbm, dst_hbm, zeros_hbm, out_hbm,
                idx_v, idx_r, ones_v, acc_out, acc_in):
    c = lax.axis_index("c")
    s = lax.axis_index("s")
    tid = c * NS + s
    for j in range(CHUNK // 16):
        ones_v[pl.ds(j * 16, 16)] = jnp.full((16,), 1.0, jnp.float32)

    @pl.when(s == 0)
    def _():
        pltpu.sync_copy(zeros_hbm, acc_out)
        pltpu.sync_copy(zeros_hbm, acc_in)

    plsc.subcore_barrier()
    base = tid * EDGES_PER_TILE

    def body(i, carry):
        e0 = base + i * CHUNK
        pltpu.sync_copy(src_hbm.at[pl.ds(e0, CHUNK)], idx_v)
        pltpu.sync_copy(ones_v, acc_out.at[idx_v], add=True)
        pltpu.sync_copy(dst_hbm.at[pl.ds(e0, CHUNK)], idx_v)
        pltpu.sync_copy(ones_v, acc_in.at[idx_v], add=True)
        return carry

    lax.fori_loop(0, NFULL, body, 0)
    e0 = base + NFULL * CHUNK
    pltpu.sync_copy(src_hbm.at[pl.ds(e0, REM)], idx_r)
    pltpu.sync_copy(ones_v.at[pl.ds(0, REM)], acc_out.at[idx_r], add=True)
    pltpu.sync_copy(dst_hbm.at[pl.ds(e0, REM)], idx_r)
    pltpu.sync_copy(ones_v.at[pl.ds(0, REM)], acc_in.at[idx_r], add=True)
    plsc.subcore_barrier()

    @pl.when(s == 0)
    def _():
        pltpu.sync_copy(acc_out, out_hbm.at[c, 0])
        pltpu.sync_copy(acc_in, out_hbm.at[c, 1])


def _make_sc_agg(D):
    @functools.partial(
        pl.kernel,
        out_type=jax.ShapeDtypeStruct((NC, N, D), jnp.float32),
        mesh=_MESH,
        scratch_types=[
            pltpu.VMEM((ACH,), jnp.int32),
            pltpu.VMEM((ACH,), jnp.int32),
            pltpu.VMEM((ACH,), jnp.int32),
            pltpu.VMEM((ACH,), jnp.int32),
            pltpu.VMEM((ACH, D), jnp.float32),
            pltpu.VMEM((ACH, D), jnp.float32),
            pltpu.VMEM_SHARED((N_PAD, D), jnp.float32),
            pltpu.SemaphoreType.DMA,
            pltpu.SemaphoreType.DMA,
        ],
    )
    def _agg(x_hbm, srcp_hbm, dstp_hbm, zeros_hbm, out_hbm,
             srcA, dstA, srcB, dstB, bufA, bufB, acc, sgA, sgB):
        c = lax.axis_index("c")
        s = lax.axis_index("s")
        tid = c * NS + s

        @pl.when(s < NS - 1)
        def _():
            pltpu.sync_copy(zeros_hbm.at[pl.ds(s * ZROWS, ZROWS)],
                            acc.at[pl.ds(s * ZROWS, ZROWS)])

        @pl.when(s == NS - 1)
        def _():
            pltpu.sync_copy(zeros_hbm.at[pl.ds((NS - 1) * ZROWS, ZROWS_LAST_PAD)],
                            acc.at[pl.ds((NS - 1) * ZROWS, ZROWS_LAST_PAD)])

        plsc.subcore_barrier()
        ebase = tid * BASE_CH * ACH

        def ld(i, sref, dref):
            pltpu.sync_copy(srcp_hbm.at[pl.ds(ebase + i * ACH, ACH)], sref)
            pltpu.sync_copy(dstp_hbm.at[pl.ds(ebase + i * ACH, ACH)], dref)

        def g_start(sref, buf, sem):
            pltpu.async_copy(x_hbm.at[sref], buf, sem)

        def g_wait(sref, buf, sem):
            pltpu.make_async_copy(x_hbm.at[sref], buf, sem).wait()

        ld(0, srcA, dstA)
        g_start(srcA, bufA, sgA)

        def body(k, carry):
            i0 = 2 * k
            ld(i0 + 1, srcB, dstB)
            g_start(srcB, bufB, sgB)
            g_wait(srcA, bufA, sgA)
            pltpu.sync_copy(bufA, acc.at[dstA], add=True)

            @pl.when(i0 + 2 < BASE_CH)
            def _():
                ld(i0 + 2, srcA, dstA)
                g_start(srcA, bufA, sgA)

            g_wait(srcB, bufB, sgB)
            pltpu.sync_copy(bufB, acc.at[dstB], add=True)
            return carry

        lax.fori_loop(0, BASE_CH // 2, body, 0)
        plsc.subcore_barrier()

        @pl.when(s < NS - 1)
        def _():
            pltpu.sync_copy(acc.at[pl.ds(s * ZROWS, ZROWS)],
                            out_hbm.at[c, pl.ds(s * ZROWS, ZROWS)])

        @pl.when(s == NS - 1)
        def _():
            pltpu.sync_copy(acc.at[pl.ds((NS - 1) * ZROWS, ZROWS_LAST)],
                            out_hbm.at[c, pl.ds((NS - 1) * ZROWS, ZROWS_LAST)])

    return _agg


_sc_agg128 = _make_sc_agg(F_HID)
_sc_agg48 = _make_sc_agg(F_OUT_PAD)


BN = 1000


def _tc_norms(degp):
    def body(d_ref, o_ref):
        d = d_ref[0] + d_ref[1]
        o_ref[...] = lax.rsqrt(jnp.where(d > 0, d, 1.0))

    return pl.pallas_call(
        body,
        out_shape=jax.ShapeDtypeStruct((2, N), jnp.float32),
    )(degp)


def _tc_layer0(feat, ns, w):
    def body(f_ref, ns_ref, w_ref, o_ref):
        o_ref[...] = jnp.dot(f_ref[...] * ns_ref[...], w_ref[...],
                             preferred_element_type=jnp.float32)

    return pl.pallas_call(
        body,
        grid=(N // BN,),
        in_specs=[
            pl.BlockSpec((BN, F_IN), lambda i: (i, 0)),
            pl.BlockSpec((BN, 1), lambda i: (i, 0)),
            pl.BlockSpec((F_IN, F_HID), lambda i: (0, 0)),
        ],
        out_specs=pl.BlockSpec((BN, F_HID), lambda i: (i, 0)),
        out_shape=jax.ShapeDtypeStruct((N, F_HID), jnp.float32),
    )(feat, ns, w)


def _tc_mid(aggp, nd, b, ns, w, d_out):
    def body(a_ref, nd_ref, b_ref, ns_ref, w_ref, o_ref):
        a = a_ref[0] + a_ref[1]
        h = jnp.maximum(a * nd_ref[...] + b_ref[...], 0.0)
        o_ref[...] = jnp.dot(h * ns_ref[...], w_ref[...],
                             preferred_element_type=jnp.float32)

    return pl.pallas_call(
        body,
        grid=(N // BN,),
        in_specs=[
            pl.BlockSpec((NC, BN, F_HID), lambda i: (0, i, 0)),
            pl.BlockSpec((BN, 1), lambda i: (i, 0)),
            pl.BlockSpec((1, F_HID), lambda i: (0, 0)),
            pl.BlockSpec((BN, 1), lambda i: (i, 0)),
            pl.BlockSpec((F_HID, d_out), lambda i: (0, 0)),
        ],
        out_specs=pl.BlockSpec((BN, d_out), lambda i: (i, 0)),
        out_shape=jax.ShapeDtypeStruct((N, d_out), jnp.float32),
    )(aggp, nd, b, ns, w)


def _tc_final(aggp, nd, b):
    def body(a_ref, nd_ref, b_ref, o_ref):
        a = a_ref[0] + a_ref[1]
        o_ref[...] = a * nd_ref[...] + b_ref[...]

    return pl.pallas_call(
        body,
        grid=(N // BN,),
        in_specs=[
            pl.BlockSpec((NC, BN, F_OUT_PAD), lambda i: (0, i, 0)),
            pl.BlockSpec((BN, 1), lambda i: (i, 0)),
            pl.BlockSpec((1, F_OUT_PAD), lambda i: (0, 0)),
        ],
        out_specs=pl.BlockSpec((BN, F_OUT_PAD), lambda i: (i, 0)),
        out_shape=jax.ShapeDtypeStruct((N, F_OUT_PAD), jnp.float32),
    )(aggp, nd, b)


def kernel(feat, edge_index, W0, b0, Wh, bh, Wo, bo):
    src = edge_index[0]
    dst = edge_index[1]
    pad_e = TOTAL_CHUNKS * ACH - E
    srcp = jnp.concatenate([src, jnp.zeros((pad_e,), jnp.int32)])
    dstp = jnp.concatenate([dst, jnp.full((pad_e,), N, jnp.int32)])
    zeros_n = jnp.zeros((N,), jnp.float32)
    zeros128 = jnp.zeros((N_PAD, F_HID), jnp.float32)
    zeros48 = jnp.zeros((N_PAD, F_OUT_PAD), jnp.float32)

    degp = _sc_degrees(src, dst, zeros_n)
    norms = _tc_norms(degp)
    ns = norms[0].reshape(N, 1)
    nd = norms[1].reshape(N, 1)

    x0 = _tc_layer0(feat, ns, W0)
    a0 = _sc_agg128(x0, srcp, dstp, zeros128)
    x1 = _tc_mid(a0, nd, b0.reshape(1, F_HID), ns, Wh, F_HID)
    a1 = _sc_agg128(x1, srcp, dstp, zeros128)

    wo_p = jnp.zeros((F_HID, F_OUT_PAD), jnp.float32).at[:, :F_OUT].set(Wo)
    bo_p = jnp.zeros((1, F_OUT_PAD), jnp.float32).at[0, :F_OUT].set(bo)
    x2 = _tc_mid(a1, nd, bh.reshape(1, F_HID), ns, wo_p, F_OUT_PAD)
    a2 = _sc_agg48(x2, srcp, dstp, zeros48)
    out = _tc_final(a2, nd, bo_p)
    return out[:, :F_OUT]


# speedup vs baseline: 2.7459x; 2.1605x over previous
import functools

import jax
import jax.numpy as jnp
from jax import lax
from jax.experimental import pallas as pl
from jax.experimental.pallas import tpu as pltpu
from jax.experimental.pallas import tpu_sc as plsc

N = 10000
E = 320000
F_IN = 128
F_HID = 128
F_OUT = 40
F_OUT_PAD = 128

NC = 2
NS = 16
NW = NC * NS
EDGES_PER_TILE = E // NW
CHUNK = 128
NFULL = EDGES_PER_TILE // CHUNK
REM = EDGES_PER_TILE - NFULL * CHUNK
ACH = 128
ZROWS = 624
ZROWS_LAST = N - (NS - 1) * ZROWS

_MESH = plsc.VectorSubcoreMesh(
    core_axis_name="c", subcore_axis_name="s", num_cores=NC, num_subcores=NS
)


@functools.partial(
    pl.kernel,
    out_type=jax.ShapeDtypeStruct((NC, 2, N), jnp.float32),
    mesh=_MESH,
    scratch_types=[
        pltpu.VMEM((CHUNK,), jnp.int32),
        pltpu.VMEM((REM,), jnp.int32),
        pltpu.VMEM((CHUNK,), jnp.float32),
        pltpu.VMEM_SHARED((N,), jnp.float32),
        pltpu.VMEM_SHARED((N,), jnp.float32),
    ],
)
def _sc_degrees(src_hbm, dst_hbm, zeros_hbm, out_hbm,
                idx_v, idx_r, ones_v, acc_out, acc_in):
    c = lax.axis_index("c")
    s = lax.axis_index("s")
    tid = c * NS + s
    for j in range(CHUNK // 16):
        ones_v[pl.ds(j * 16, 16)] = jnp.full((16,), 1.0, jnp.float32)

    @pl.when(s == 0)
    def _():
        pltpu.sync_copy(zeros_hbm, acc_out)
        pltpu.sync_copy(zeros_hbm, acc_in)

    plsc.subcore_barrier()
    base = tid * EDGES_PER_TILE

    def body(i, carry):
        e0 = base + i * CHUNK
        pltpu.sync_copy(src_hbm.at[pl.ds(e0, CHUNK)], idx_v)
        pltpu.sync_copy(ones_v, acc_out.at[idx_v], add=True)
        pltpu.sync_copy(dst_hbm.at[pl.ds(e0, CHUNK)], idx_v)
        pltpu.sync_copy(ones_v, acc_in.at[idx_v], add=True)
        return carry

    lax.fori_loop(0, NFULL, body, 0)
    e0 = base + NFULL * CHUNK
    pltpu.sync_copy(src_hbm.at[pl.ds(e0, REM)], idx_r)
    pltpu.sync_copy(ones_v.at[pl.ds(0, REM)], acc_out.at[idx_r], add=True)
    pltpu.sync_copy(dst_hbm.at[pl.ds(e0, REM)], idx_r)
    pltpu.sync_copy(ones_v.at[pl.ds(0, REM)], acc_in.at[idx_r], add=True)
    plsc.subcore_barrier()

    @pl.when(s == 0)
    def _():
        pltpu.sync_copy(acc_out, out_hbm.at[c, 0])
        pltpu.sync_copy(acc_in, out_hbm.at[c, 1])


def _make_sc_agg(D):
    @functools.partial(
        pl.kernel,
        out_type=jax.ShapeDtypeStruct((NC, N, D), jnp.float32),
        mesh=_MESH,
        scratch_types=[
            pltpu.VMEM((ACH,), jnp.int32),
            pltpu.VMEM((ACH,), jnp.int32),
            pltpu.VMEM((ACH,), jnp.int32),
            pltpu.VMEM((ACH,), jnp.int32),
            pltpu.VMEM((REM,), jnp.int32),
            pltpu.VMEM((REM,), jnp.int32),
            pltpu.VMEM((ACH, D), jnp.float32),
            pltpu.VMEM((ACH, D), jnp.float32),
            pltpu.VMEM_SHARED((N, D), jnp.float32),
            pltpu.SemaphoreType.DMA,
            pltpu.SemaphoreType.DMA,
        ],
    )
    def _agg(x_hbm, src_hbm, dst_hbm, zeros_hbm, out_hbm,
             srcA, dstA, srcB, dstB, srcR, dstR, bufA, bufB, acc, sgA, sgB):
        c = lax.axis_index("c")
        s = lax.axis_index("s")
        tid = c * NS + s

        @pl.when(s < NS - 1)
        def _():
            pltpu.sync_copy(zeros_hbm.at[pl.ds(s * ZROWS, ZROWS)],
                            acc.at[pl.ds(s * ZROWS, ZROWS)])

        @pl.when(s == NS - 1)
        def _():
            pltpu.sync_copy(zeros_hbm.at[pl.ds((NS - 1) * ZROWS, ZROWS_LAST)],
                            acc.at[pl.ds((NS - 1) * ZROWS, ZROWS_LAST)])

        plsc.subcore_barrier()
        ebase = tid * EDGES_PER_TILE

        def ld(i, sref, dref):
            pltpu.sync_copy(src_hbm.at[pl.ds(ebase + i * ACH, ACH)], sref)
            pltpu.sync_copy(dst_hbm.at[pl.ds(ebase + i * ACH, ACH)], dref)

        def g_start(sref, buf, sem):
            pltpu.async_copy(x_hbm.at[sref], buf, sem)

        def g_wait(sref, buf, sem):
            pltpu.make_async_copy(x_hbm.at[sref], buf, sem).wait()

        ld(0, srcA, dstA)
        g_start(srcA, bufA, sgA)

        def body(k, carry):
            i0 = 2 * k
            ld(i0 + 1, srcB, dstB)
            g_start(srcB, bufB, sgB)
            g_wait(srcA, bufA, sgA)
            pltpu.sync_copy(bufA, acc.at[dstA], add=True)

            @pl.when(i0 + 2 < NFULL)
            def _():
                ld(i0 + 2, srcA, dstA)
                g_start(srcA, bufA, sgA)

            g_wait(srcB, bufB, sgB)
            pltpu.sync_copy(bufB, acc.at[dstB], add=True)
            return carry

        lax.fori_loop(0, NFULL // 2, body, 0)

        e0 = ebase + NFULL * ACH
        pltpu.sync_copy(src_hbm.at[pl.ds(e0, REM)], srcR)
        pltpu.sync_copy(x_hbm.at[srcR], bufA.at[pl.ds(0, REM)])
        pltpu.sync_copy(dst_hbm.at[pl.ds(e0, REM)], dstR)
        pltpu.sync_copy(bufA.at[pl.ds(0, REM)], acc.at[dstR], add=True)
        plsc.subcore_barrier()

        @pl.when(s < NS - 1)
        def _():
            pltpu.sync_copy(acc.at[pl.ds(s * ZROWS, ZROWS)],
                            out_hbm.at[c, pl.ds(s * ZROWS, ZROWS)])

        @pl.when(s == NS - 1)
        def _():
            pltpu.sync_copy(acc.at[pl.ds((NS - 1) * ZROWS, ZROWS_LAST)],
                            out_hbm.at[c, pl.ds((NS - 1) * ZROWS, ZROWS_LAST)])

    return _agg


_sc_agg128 = _make_sc_agg(F_HID)
_sc_agg48 = _make_sc_agg(F_OUT_PAD)


BN = 1000


def _tc_norms(degp):
    def body(d_ref, o_ref):
        d = d_ref[0] + d_ref[1]
        o_ref[...] = lax.rsqrt(jnp.where(d > 0, d, 1.0))

    return pl.pallas_call(
        body,
        out_shape=jax.ShapeDtypeStruct((2, N), jnp.float32),
    )(degp)


def _tc_layer0(feat, ns, w):
    def body(f_ref, ns_ref, w_ref, o_ref):
        o_ref[...] = jnp.dot(f_ref[...] * ns_ref[...], w_ref[...],
                             preferred_element_type=jnp.float32)

    return pl.pallas_call(
        body,
        grid=(N // BN,),
        in_specs=[
            pl.BlockSpec((BN, F_IN), lambda i: (i, 0)),
            pl.BlockSpec((BN, 1), lambda i: (i, 0)),
            pl.BlockSpec((F_IN, F_HID), lambda i: (0, 0)),
        ],
        out_specs=pl.BlockSpec((BN, F_HID), lambda i: (i, 0)),
        out_shape=jax.ShapeDtypeStruct((N, F_HID), jnp.float32),
    )(feat, ns, w)


def _tc_mid(aggp, nd, b, ns, w, d_out):
    def body(a_ref, nd_ref, b_ref, ns_ref, w_ref, o_ref):
        a = a_ref[0] + a_ref[1]
        h = jnp.maximum(a * nd_ref[...] + b_ref[...], 0.0)
        o_ref[...] = jnp.dot(h * ns_ref[...], w_ref[...],
                             preferred_element_type=jnp.float32)

    return pl.pallas_call(
        body,
        grid=(N // BN,),
        in_specs=[
            pl.BlockSpec((NC, BN, F_HID), lambda i: (0, i, 0)),
            pl.BlockSpec((BN, 1), lambda i: (i, 0)),
            pl.BlockSpec((1, F_HID), lambda i: (0, 0)),
            pl.BlockSpec((BN, 1), lambda i: (i, 0)),
            pl.BlockSpec((F_HID, d_out), lambda i: (0, 0)),
        ],
        out_specs=pl.BlockSpec((BN, d_out), lambda i: (i, 0)),
        out_shape=jax.ShapeDtypeStruct((N, d_out), jnp.float32),
    )(aggp, nd, b, ns, w)


def _tc_final(aggp, nd, b):
    def body(a_ref, nd_ref, b_ref, o_ref):
        a = a_ref[0] + a_ref[1]
        o_ref[...] = a * nd_ref[...] + b_ref[...]

    return pl.pallas_call(
        body,
        grid=(N // BN,),
        in_specs=[
            pl.BlockSpec((NC, BN, F_OUT_PAD), lambda i: (0, i, 0)),
            pl.BlockSpec((BN, 1), lambda i: (i, 0)),
            pl.BlockSpec((1, F_OUT_PAD), lambda i: (0, 0)),
        ],
        out_specs=pl.BlockSpec((BN, F_OUT_PAD), lambda i: (i, 0)),
        out_shape=jax.ShapeDtypeStruct((N, F_OUT_PAD), jnp.float32),
    )(aggp, nd, b)


def kernel(feat, edge_index, W0, b0, Wh, bh, Wo, bo):
    src = edge_index[0]
    dst = edge_index[1]
    zeros_n = jnp.zeros((N,), jnp.float32)
    zeros128 = jnp.zeros((N, F_HID), jnp.float32)
    zeros48 = jnp.zeros((N, F_OUT_PAD), jnp.float32)

    degp = _sc_degrees(src, dst, zeros_n)
    norms = _tc_norms(degp)
    ns = norms[0].reshape(N, 1)
    nd = norms[1].reshape(N, 1)

    x0 = _tc_layer0(feat, ns, W0)
    a0 = _sc_agg128(x0, src, dst, zeros128)
    x1 = _tc_mid(a0, nd, b0.reshape(1, F_HID), ns, Wh, F_HID)
    a1 = _sc_agg128(x1, src, dst, zeros128)

    wo_p = jnp.zeros((F_HID, F_OUT_PAD), jnp.float32).at[:, :F_OUT].set(Wo)
    bo_p = jnp.zeros((1, F_OUT_PAD), jnp.float32).at[0, :F_OUT].set(bo)
    x2 = _tc_mid(a1, nd, bh.reshape(1, F_HID), ns, wo_p, F_OUT_PAD)
    a2 = _sc_agg48(x2, src, dst, zeros48)
    out = _tc_final(a2, nd, bo_p)
    return out[:, :F_OUT]


# speedup vs baseline: 2.7516x; 1.0021x over previous
import functools

import jax
import jax.numpy as jnp
from jax import lax
from jax.experimental import pallas as pl
from jax.experimental.pallas import tpu as pltpu
from jax.experimental.pallas import tpu_sc as plsc

N = 10000
E = 320000
F_IN = 128
F_HID = 128
F_OUT = 40
F_OUT_PAD = 128

NC = 2
NS = 16
NW = NC * NS
EDGES_PER_TILE = E // NW
CHUNK = 128
NFULL = EDGES_PER_TILE // CHUNK
REM = EDGES_PER_TILE - NFULL * CHUNK
ACH = 128
ZROWS = 624
ZROWS_LAST = N - (NS - 1) * ZROWS

_MESH = plsc.VectorSubcoreMesh(
    core_axis_name="c", subcore_axis_name="s", num_cores=NC, num_subcores=NS
)


@functools.partial(
    pl.kernel,
    out_type=jax.ShapeDtypeStruct((NC, 2, N), jnp.float32),
    mesh=_MESH,
    scratch_types=[
        pltpu.VMEM((CHUNK,), jnp.int32),
        pltpu.VMEM((REM,), jnp.int32),
        pltpu.VMEM((CHUNK,), jnp.float32),
        pltpu.VMEM_SHARED((N,), jnp.float32),
        pltpu.VMEM_SHARED((N,), jnp.float32),
    ],
)
def _sc_degrees(src_hbm, dst_hbm, zeros_hbm, out_hbm,
                idx_v, idx_r, ones_v, acc_out, acc_in):
    c = lax.axis_index("c")
    s = lax.axis_index("s")
    tid = c * NS + s
    for j in range(CHUNK // 16):
        ones_v[pl.ds(j * 16, 16)] = jnp.full((16,), 1.0, jnp.float32)

    @pl.when(s == 0)
    def _():
        pltpu.sync_copy(zeros_hbm, acc_out)
        pltpu.sync_copy(zeros_hbm, acc_in)

    plsc.subcore_barrier()
    base = tid * EDGES_PER_TILE

    def body(i, carry):
        e0 = base + i * CHUNK
        pltpu.sync_copy(src_hbm.at[pl.ds(e0, CHUNK)], idx_v)
        pltpu.sync_copy(ones_v, acc_out.at[idx_v], add=True)
        pltpu.sync_copy(dst_hbm.at[pl.ds(e0, CHUNK)], idx_v)
        pltpu.sync_copy(ones_v, acc_in.at[idx_v], add=True)
        return carry

    lax.fori_loop(0, NFULL, body, 0)
    e0 = base + NFULL * CHUNK
    pltpu.sync_copy(src_hbm.at[pl.ds(e0, REM)], idx_r)
    pltpu.sync_copy(ones_v.at[pl.ds(0, REM)], acc_out.at[idx_r], add=True)
    pltpu.sync_copy(dst_hbm.at[pl.ds(e0, REM)], idx_r)
    pltpu.sync_copy(ones_v.at[pl.ds(0, REM)], acc_in.at[idx_r], add=True)
    plsc.subcore_barrier()

    @pl.when(s == 0)
    def _():
        pltpu.sync_copy(acc_out, out_hbm.at[c, 0])
        pltpu.sync_copy(acc_in, out_hbm.at[c, 1])


def _make_sc_agg(D):
    @functools.partial(
        pl.kernel,
        out_type=jax.ShapeDtypeStruct((NC, N, D), jnp.float32),
        mesh=_MESH,
        scratch_types=[
            pltpu.VMEM((ACH,), jnp.int32),
            pltpu.VMEM((ACH,), jnp.int32),
            pltpu.VMEM((ACH,), jnp.int32),
            pltpu.VMEM((ACH,), jnp.int32),
            pltpu.VMEM((REM,), jnp.int32),
            pltpu.VMEM((REM,), jnp.int32),
            pltpu.VMEM((ACH, D), jnp.float32),
            pltpu.VMEM((ACH, D), jnp.float32),
            pltpu.VMEM_SHARED((N, D), jnp.float32),
            pltpu.SemaphoreType.DMA,
            pltpu.SemaphoreType.DMA,
        ],
    )
    def _agg(x_hbm, src_hbm, dst_hbm, zeros_hbm, out_hbm,
             srcA, dstA, srcB, dstB, srcR, dstR, bufA, bufB, acc, sgA, sgB):
        c = lax.axis_index("c")
        s = lax.axis_index("s")
        tid = c * NS + s

        @pl.when(s < NS - 1)
        def _():
            pltpu.sync_copy(zeros_hbm.at[pl.ds(s * ZROWS, ZROWS)],
                            acc.at[pl.ds(s * ZROWS, ZROWS)])

        @pl.when(s == NS - 1)
        def _():
            pltpu.sync_copy(zeros_hbm.at[pl.ds((NS - 1) * ZROWS, ZROWS_LAST)],
                            acc.at[pl.ds((NS - 1) * ZROWS, ZROWS_LAST)])

        plsc.subcore_barrier()
        ebase = tid * EDGES_PER_TILE

        def ld(i, sref, dref):
            pltpu.sync_copy(src_hbm.at[pl.ds(ebase + i * ACH, ACH)], sref)
            pltpu.sync_copy(dst_hbm.at[pl.ds(ebase + i * ACH, ACH)], dref)

        def g_start(sref, buf, sem):
            pltpu.async_copy(x_hbm.at[sref], buf, sem)

        def g_wait(sref, buf, sem):
            pltpu.make_async_copy(x_hbm.at[sref], buf, sem).wait()

        ld(0, srcA, dstA)
        g_start(srcA, bufA, sgA)

        def body(k, carry):
            i0 = 2 * k
            ld(i0 + 1, srcB, dstB)
            g_start(srcB, bufB, sgB)
            g_wait(srcA, bufA, sgA)
            pltpu.sync_copy(bufA, acc.at[dstA], add=True)

            @pl.when(i0 + 2 < NFULL)
            def _():
                ld(i0 + 2, srcA, dstA)
                g_start(srcA, bufA, sgA)

            g_wait(srcB, bufB, sgB)
            pltpu.sync_copy(bufB, acc.at[dstB], add=True)
            return carry

        lax.fori_loop(0, NFULL // 2, body, 0)

        e0 = ebase + NFULL * ACH
        pltpu.sync_copy(src_hbm.at[pl.ds(e0, REM)], srcR)
        pltpu.sync_copy(x_hbm.at[srcR], bufA.at[pl.ds(0, REM)])
        pltpu.sync_copy(dst_hbm.at[pl.ds(e0, REM)], dstR)
        pltpu.sync_copy(bufA.at[pl.ds(0, REM)], acc.at[dstR], add=True)
        plsc.subcore_barrier()

        @pl.when(s < NS - 1)
        def _():
            pltpu.sync_copy(acc.at[pl.ds(s * ZROWS, ZROWS)],
                            out_hbm.at[c, pl.ds(s * ZROWS, ZROWS)])

        @pl.when(s == NS - 1)
        def _():
            pltpu.sync_copy(acc.at[pl.ds((NS - 1) * ZROWS, ZROWS_LAST)],
                            out_hbm.at[c, pl.ds((NS - 1) * ZROWS, ZROWS_LAST)])

    return _agg


_sc_agg128 = _make_sc_agg(F_HID)
_sc_agg_out = _make_sc_agg(F_OUT_PAD)


BN = 1000


def _tc_norms(degp):
    def body(d_ref, o_ref):
        d = d_ref[0] + d_ref[1]
        o_ref[...] = lax.rsqrt(jnp.where(d > 0, d, 1.0))

    return pl.pallas_call(
        body,
        out_shape=jax.ShapeDtypeStruct((2, N), jnp.float32),
    )(degp)


def _tc_layer0(feat, ns, w):
    def body(f_ref, ns_ref, w_ref, o_ref):
        o_ref[...] = jnp.dot(f_ref[...] * ns_ref[...], w_ref[...],
                             preferred_element_type=jnp.float32)

    return pl.pallas_call(
        body,
        grid=(N // BN,),
        in_specs=[
            pl.BlockSpec((BN, F_IN), lambda i: (i, 0)),
            pl.BlockSpec((BN, 1), lambda i: (i, 0)),
            pl.BlockSpec((F_IN, F_HID), lambda i: (0, 0)),
        ],
        out_specs=pl.BlockSpec((BN, F_HID), lambda i: (i, 0)),
        out_shape=jax.ShapeDtypeStruct((N, F_HID), jnp.float32),
    )(feat, ns, w)


def _tc_mid(aggp, nd, b, ns, w, d_out):
    def body(a_ref, nd_ref, b_ref, ns_ref, w_ref, o_ref):
        a = a_ref[0] + a_ref[1]
        h = jnp.maximum(a * nd_ref[...] + b_ref[...], 0.0)
        o_ref[...] = jnp.dot(h * ns_ref[...], w_ref[...],
                             preferred_element_type=jnp.float32)

    return pl.pallas_call(
        body,
        grid=(N // BN,),
        in_specs=[
            pl.BlockSpec((NC, BN, F_HID), lambda i: (0, i, 0)),
            pl.BlockSpec((BN, 1), lambda i: (i, 0)),
            pl.BlockSpec((1, F_HID), lambda i: (0, 0)),
            pl.BlockSpec((BN, 1), lambda i: (i, 0)),
            pl.BlockSpec((F_HID, d_out), lambda i: (0, 0)),
        ],
        out_specs=pl.BlockSpec((BN, d_out), lambda i: (i, 0)),
        out_shape=jax.ShapeDtypeStruct((N, d_out), jnp.float32),
    )(aggp, nd, b, ns, w)


def _tc_final(aggp, nd, b):
    def body(a_ref, nd_ref, b_ref, o_ref):
        a = a_ref[0] + a_ref[1]
        o_ref[...] = a * nd_ref[...] + b_ref[...]

    return pl.pallas_call(
        body,
        grid=(N // BN,),
        in_specs=[
            pl.BlockSpec((NC, BN, F_OUT_PAD), lambda i: (0, i, 0)),
            pl.BlockSpec((BN, 1), lambda i: (i, 0)),
            pl.BlockSpec((1, F_OUT_PAD), lambda i: (0, 0)),
        ],
        out_specs=pl.BlockSpec((BN, F_OUT_PAD), lambda i: (i, 0)),
        out_shape=jax.ShapeDtypeStruct((N, F_OUT_PAD), jnp.float32),
    )(aggp, nd, b)


def kernel(feat, edge_index, W0, b0, Wh, bh, Wo, bo):
    src = edge_index[0]
    dst = edge_index[1]
    zeros_n = jnp.zeros((N,), jnp.float32)
    zeros128 = jnp.zeros((N, F_HID), jnp.float32)
    zeros48 = jnp.zeros((N, F_OUT_PAD), jnp.float32)

    degp = _sc_degrees(src, dst, zeros_n)
    norms = _tc_norms(degp)
    ns = norms[0].reshape(N, 1)
    nd = norms[1].reshape(N, 1)

    x0 = _tc_layer0(feat, ns, W0)
    a0 = _sc_agg128(x0, src, dst, zeros128)
    x1 = _tc_mid(a0, nd, b0.reshape(1, F_HID), ns, Wh, F_HID)
    a1 = _sc_agg128(x1, src, dst, zeros128)

    wo_p = jnp.zeros((F_HID, F_OUT_PAD), jnp.float32).at[:, :F_OUT].set(Wo)
    bo_p = jnp.zeros((1, F_OUT_PAD), jnp.float32).at[0, :F_OUT].set(bo)
    x2 = _tc_mid(a1, nd, bh.reshape(1, F_HID), ns, wo_p, F_OUT_PAD)
    a2 = _sc_agg_out(x2, src, dst, zeros48)
    out = _tc_final(a2, nd, bo_p)
    return out[:, :F_OUT]
